# baseline (speedup 1.0000x reference)
import functools

import jax
import jax.numpy as jnp
from jax import lax
from jax.experimental import pallas as pl
from jax.experimental.pallas import tpu as pltpu
from jax.experimental.pallas import tpu_sc as plsc

B, T, F, D = 4096, 50, 26, 32
V = 100000
NF = 13

_info = plsc.get_sparse_core_info()
NC, NS = _info.num_cores, _info.num_subcores
NW = NC * NS
CB = B // NW
NRING = 4


def _sc_body(x_hbm, tab_hbm, w_hbm, b_hbm, out_hbm,
             xall, idx_v, gathv, outc, outt, w_v, b_v,
             sem_g, sem_oc, sem_ot):
    wid = lax.axis_index("s") * NC + lax.axis_index("c")
    c0 = wid * CB
    pltpu.sync_copy(w_hbm, w_v)
    pltpu.sync_copy(b_hbm, b_v)
    iota = lax.iota(jnp.int32, 16)
    iota_d = iota * D

    def unit(t, carry):
        pltpu.sync_copy(x_hbm.at[:, pl.ds(t, 1), pl.ds(c0, CB)], xall)

        for j in range(NF):
            for g in range(CB // 16):
                xg = xall[2 * j, 0, pl.ds(g * 16, 16)]
                idx_v[j, pl.ds(g * 16, 16)] = xg.astype(jnp.int32)
        gd = [pltpu.async_copy(tab_hbm.at[j].at[idx_v.at[j]], gathv.at[j],
                               sem_g)
              for j in range(NF)]

        ocd = []
        for j in range(NF):
            rb = j % NRING
            if j >= NRING:
                ocd[j - NRING].wait()

            def silu_d(d, _, j=j, rb=rb):
                wd = plsc.load_gather(w_v, [jnp.full((16,), j * D + d,
                                                     jnp.int32)])
                bd = plsc.load_gather(b_v, [jnp.full((16,), j * D + d,
                                                     jnp.int32)])
                for g in range(CB // 16):
                    xg = xall[2 * j + 1, 0, pl.ds(g * 16, 16)]
                    v = xg * wd + bd
                    outc[rb, 0, d, pl.ds(g * 16, 16)] = v / (1.0 + jnp.exp(-v))
                return _

            lax.fori_loop(0, D, silu_d, 0)
            ocd.append(pltpu.async_copy(
                outc.at[rb], out_hbm.at[pl.ds(t, 1), 2 * j + 1, :,
                                        pl.ds(c0, CB)], sem_oc))

        otd = []
        for j in range(NF):
            gd[j].wait()
            rb = j % NRING
            if j >= NRING:
                otd[j - NRING].wait()

            def trans_d(d, _, j=j, rb=rb):
                col = jnp.full((16,), d, jnp.int32)
                for g in range(CB // 16):
                    row = plsc.load_gather(gathv.at[j],
                                           [g * 16 + iota, col])
                    outt[rb, 0, d, pl.ds(g * 16, 16)] = row
                return _

            lax.fori_loop(0, D, trans_d, 0)
            otd.append(pltpu.async_copy(
                outt.at[rb], out_hbm.at[pl.ds(t, 1), 2 * j, :,
                                        pl.ds(c0, CB)], sem_ot))

        for dsc in ocd[-NRING:]:
            dsc.wait()
        for dsc in otd[-NRING:]:
            dsc.wait()
        return carry

    lax.fori_loop(0, T, unit, 0)


@jax.jit
def kernel(x, tables, W, b):
    xr = jnp.transpose(x, (2, 1, 0))
    run = pl.kernel(
        _sc_body,
        out_type=jax.ShapeDtypeStruct((T, F, D, B), jnp.float32),
        mesh=plsc.VectorSubcoreMesh(core_axis_name="c", subcore_axis_name="s"),
        compiler_params=pltpu.CompilerParams(
            needs_layout_passes=False, use_tc_tiling_on_sc=False),
        scratch_types=[
            pltpu.VMEM((F, 1, CB), jnp.float32),
            pltpu.VMEM((NF, CB), jnp.int32),
            pltpu.VMEM((NF, CB, D), jnp.float32),
            pltpu.VMEM((NRING, 1, D, CB), jnp.float32),
            pltpu.VMEM((NRING, 1, D, CB), jnp.float32),
            pltpu.VMEM((NF * D,), jnp.float32),
            pltpu.VMEM((NF * D,), jnp.float32),
            pltpu.SemaphoreType.DMA,
            pltpu.SemaphoreType.DMA,
            pltpu.SemaphoreType.DMA,
        ],
    )
    outp = run(xr, tables, W.reshape(-1), b.reshape(-1))
    return jnp.transpose(outp, (3, 0, 1, 2))


# speedup vs baseline: 1.9906x; 1.9906x over previous
import functools

import jax
import jax.numpy as jnp
from jax import lax
from jax.experimental import pallas as pl
from jax.experimental.pallas import tpu as pltpu
from jax.experimental.pallas import tpu_sc as plsc

B, T, F, D = 4096, 50, 26, 32
V = 100000
NF = 13

_info = plsc.get_sparse_core_info()
NC, NS = _info.num_cores, _info.num_subcores
NW = NC * NS
CB = B // NW
NRING = 4


def _sc_body(x_hbm, tab_hbm, out_hbm, xall, idx_v, gathv, outt,
             sem_g, sem_ot):
    wid = lax.axis_index("s") * NC + lax.axis_index("c")
    c0 = wid * CB
    iota = lax.iota(jnp.int32, 16)

    def unit(t, carry):
        pltpu.sync_copy(x_hbm.at[:, pl.ds(t, 1), pl.ds(c0, CB)], xall)

        for j in range(NF):
            for g in range(CB // 16):
                xg = xall[2 * j, 0, pl.ds(g * 16, 16)]
                idx_v[j, pl.ds(g * 16, 16)] = xg.astype(jnp.int32)
        gd = [pltpu.async_copy(tab_hbm.at[j].at[idx_v.at[j]], gathv.at[j],
                               sem_g)
              for j in range(NF)]

        otd = []
        for j in range(NF):
            gd[j].wait()
            rb = j % NRING
            if j >= NRING:
                otd[j - NRING].wait()

            def trans_d(d, _, j=j, rb=rb):
                col = jnp.full((16,), d, jnp.int32)
                for g in range(CB // 16):
                    row = plsc.load_gather(gathv.at[j],
                                           [g * 16 + iota, col])
                    outt[rb, 0, d, pl.ds(g * 16, 16)] = row
                return _

            lax.fori_loop(0, D, trans_d, 0)
            otd.append(pltpu.async_copy(
                outt.at[rb], out_hbm.at[pl.ds(t, 1), 2 * j, :,
                                        pl.ds(c0, CB)], sem_ot))

        for dsc in otd[-NRING:]:
            dsc.wait()
        return carry

    lax.fori_loop(0, T, unit, 0)


def _tc_body(x_ref, w_ref, b_ref, donor_ref, out_ref):
    for j in range(NF):
        xv = x_ref[0, 2 * j + 1, :]
        wv = w_ref[:, j:j + 1]
        bv = b_ref[:, j:j + 1]
        v = xv[None, :] * wv + bv
        out_ref[0, j, 0, :, :] = v * jax.nn.sigmoid(v)


@jax.jit
def kernel(x, tables, W, b):
    xr = jnp.transpose(x, (2, 1, 0))

    sc_run = pl.kernel(
        _sc_body,
        out_type=jax.ShapeDtypeStruct((T, F, D, B), jnp.float32),
        mesh=plsc.VectorSubcoreMesh(core_axis_name="c", subcore_axis_name="s"),
        compiler_params=pltpu.CompilerParams(
            needs_layout_passes=False, use_tc_tiling_on_sc=False),
        scratch_types=[
            pltpu.VMEM((F, 1, CB), jnp.float32),
            pltpu.VMEM((NF, CB), jnp.int32),
            pltpu.VMEM((NF, CB, D), jnp.float32),
            pltpu.VMEM((NRING, 1, D, CB), jnp.float32),
            pltpu.SemaphoreType.DMA,
            pltpu.SemaphoreType.DMA,
        ],
    )
    outp = sc_run(xr, tables)

    xt = jnp.transpose(x, (1, 2, 0))
    outp = pl.pallas_call(
        _tc_body,
        grid=(T,),
        in_specs=[
            pl.BlockSpec((1, F, B), lambda t: (t, 0, 0)),
            pl.BlockSpec((D, NF), lambda t: (0, 0)),
            pl.BlockSpec((D, NF), lambda t: (0, 0)),
            pl.BlockSpec(memory_space=pl.ANY),
        ],
        out_specs=pl.BlockSpec((1, NF, 1, D, B), lambda t: (t, 0, 1, 0, 0)),
        out_shape=jax.ShapeDtypeStruct((T, NF, 2, D, B), jnp.float32),
        input_output_aliases={3: 0},
    )(xt, W.T, b.T, outp.reshape(T, NF, 2, D, B))
    outp = outp.reshape(T, F, D, B)

    return jnp.transpose(outp, (3, 0, 1, 2))


# speedup vs baseline: 2.0275x; 1.0185x over previous
import functools

import jax
import jax.numpy as jnp
from jax import lax
from jax.experimental import pallas as pl
from jax.experimental.pallas import tpu as pltpu
from jax.experimental.pallas import tpu_sc as plsc

B, T, F, D = 4096, 50, 26, 32
V = 100000
NF = 13

_info = plsc.get_sparse_core_info()
NC, NS = _info.num_cores, _info.num_subcores
NW = NC * NS
CB = B // NW
NRING = 4


def _sc_body(x_hbm, tab_hbm, out_hbm, xall, idx_v, gathv, outt,
             sem_x, sem_g, sem_ot):
    wid = lax.axis_index("s") * NC + lax.axis_index("c")
    c0 = wid * CB
    iota = lax.iota(jnp.int32, 16)

    def x_copy(t):
        return pltpu.make_async_copy(
            x_hbm.at[:, pl.ds(t, 1), pl.ds(c0, CB)], xall, sem_x)

    def gather_copy(tp, j):
        s = lax.rem(tp, 2)
        return pltpu.make_async_copy(
            tab_hbm.at[j].at[idx_v.at[s, j]], gathv.at[s, j], sem_g.at[s])

    def out_wait(rb):
        pltpu.make_async_copy(
            outt.at[rb], out_hbm.at[pl.ds(0, 1), 0, 0, :, pl.ds(c0, CB)],
            sem_ot.at[rb]).wait()

    def phase_a(t):
        x_copy(t).wait()
        s = lax.rem(t, 2)
        for j in range(NF):
            for g in range(CB // 16):
                xg = xall[2 * j, 0, pl.ds(g * 16, 16)]
                idx_v[s, j, pl.ds(g * 16, 16)] = xg.astype(jnp.int32)
        @pl.when(t < T - 1)
        def _():
            x_copy(t + 1).start()
        for j in range(NF):
            gather_copy(t, j).start()

    def phase_b(tp, first):
        for j in range(NF):
            gather_copy(tp, j).wait()
            rb = j % NRING
            if first:
                if j >= NRING:
                    out_wait(rb)
            else:
                out_wait(rb)

            def trans_d(d, _, j=j, rb=rb):
                col = jnp.full((16,), d, jnp.int32)
                s = lax.rem(tp, 2)
                for g in range(CB // 16):
                    row = plsc.load_gather(gathv.at[s, j],
                                           [g * 16 + iota, col])
                    outt[rb, 0, d, pl.ds(g * 16, 16)] = row
                return _

            lax.fori_loop(0, D, trans_d, 0)
            pltpu.make_async_copy(
                outt.at[rb], out_hbm.at[pl.ds(tp, 1), j, 0, :,
                                        pl.ds(c0, CB)], sem_ot.at[rb]).start()

    x_copy(0).start()

    def unit0(t, carry):
        phase_a(t)

        @pl.when(t == 1)
        def _():
            phase_b(0, first=True)

        @pl.when(t > 1)
        def _():
            phase_b(t - 1, first=False)
        return carry

    lax.fori_loop(0, T, unit0, 0)
    phase_b(T - 1, first=False)
    for rb in range(NRING):
        out_wait(rb)


def _tc_body(x_ref, w_ref, b_ref, donor_ref, out_ref):
    for j in range(NF):
        xv = x_ref[0, 2 * j + 1, :]
        wv = w_ref[:, j:j + 1]
        bv = b_ref[:, j:j + 1]
        v = xv[None, :] * wv + bv
        out_ref[0, j, 0, :, :] = v * jax.nn.sigmoid(v)


@jax.jit
def kernel(x, tables, W, b):
    xr = jnp.transpose(x, (2, 1, 0))

    sc_run = pl.kernel(
        _sc_body,
        out_type=jax.ShapeDtypeStruct((T, NF, 2, D, B), jnp.float32),
        mesh=plsc.VectorSubcoreMesh(core_axis_name="c", subcore_axis_name="s"),
        compiler_params=pltpu.CompilerParams(
            needs_layout_passes=False, use_tc_tiling_on_sc=False),
        scratch_types=[
            pltpu.VMEM((F, 1, CB), jnp.float32),
            pltpu.VMEM((2, NF, CB), jnp.int32),
            pltpu.VMEM((2, NF, CB, D), jnp.float32),
            pltpu.VMEM((NRING, 1, D, CB), jnp.float32),
            pltpu.SemaphoreType.DMA,
            pltpu.SemaphoreType.DMA((2,)),
            pltpu.SemaphoreType.DMA((NRING,)),
        ],
    )
    outp = sc_run(xr, tables)

    xt = jnp.transpose(x, (1, 2, 0))
    outp = pl.pallas_call(
        _tc_body,
        grid=(T,),
        in_specs=[
            pl.BlockSpec((1, F, B), lambda t: (t, 0, 0)),
            pl.BlockSpec((D, NF), lambda t: (0, 0)),
            pl.BlockSpec((D, NF), lambda t: (0, 0)),
            pl.BlockSpec(memory_space=pl.ANY),
        ],
        out_specs=pl.BlockSpec((1, NF, 1, D, B), lambda t: (t, 0, 1, 0, 0)),
        out_shape=jax.ShapeDtypeStruct((T, NF, 2, D, B), jnp.float32),
        input_output_aliases={3: 0},
    )(xt, W.T, b.T, outp)
    outp = outp.reshape(T, F, D, B)

    return jnp.transpose(outp, (3, 0, 1, 2))


# speedup vs baseline: 2.2561x; 1.1127x over previous
import functools

import jax
import jax.numpy as jnp
from jax import lax
from jax.experimental import pallas as pl
from jax.experimental.pallas import tpu as pltpu
from jax.experimental.pallas import tpu_sc as plsc

B, T, F, D = 4096, 50, 26, 32
V = 100000
NF = 13

_info = plsc.get_sparse_core_info()
NC, NS = _info.num_cores, _info.num_subcores
NW = NC * NS
CB = B // NW
NRING = 4


def _sc_body(x_hbm, tab_hbm, out_hbm, xall, idx_v, gathv, outt,
             sem_x, sem_g, sem_ot):
    wid = lax.axis_index("s") * NC + lax.axis_index("c")
    c0 = wid * CB
    iota = lax.iota(jnp.int32, 16)

    def x_copy(t):
        return pltpu.make_async_copy(
            x_hbm.at[:, pl.ds(t, 1), pl.ds(c0, CB)], xall, sem_x)

    def gather_copy(tp, j):
        s = lax.rem(tp, 2)
        return pltpu.make_async_copy(
            tab_hbm.at[j].at[idx_v.at[s, j]], gathv.at[s, j], sem_g.at[s])

    def out_wait(rb):
        pltpu.make_async_copy(
            outt.at[rb], out_hbm.at[pl.ds(0, 1), 0, 0, :, pl.ds(c0, CB)],
            sem_ot.at[rb]).wait()

    def phase_a(t):
        x_copy(t).wait()
        s = lax.rem(t, 2)
        for j in range(NF):
            for g in range(CB // 16):
                xg = xall[2 * j, 0, pl.ds(g * 16, 16)]
                idx_v[s, j, pl.ds(g * 16, 16)] = xg.astype(jnp.int32)
        @pl.when(t < T - 1)
        def _():
            x_copy(t + 1).start()
        for j in range(NF):
            gather_copy(t, j).start()

    def phase_b(tp, first):
        for j in range(NF):
            gather_copy(tp, j).wait()
            rb = j % NRING
            if first:
                if j >= NRING:
                    out_wait(rb)
            else:
                out_wait(rb)

            s = lax.rem(tp, 2)

            def trans_d(d, _, j=j, rb=rb, s=s):
                col = jnp.full((16,), d, jnp.int32)
                rows = [plsc.load_gather(gathv.at[s, j],
                                         [g * 16 + iota, col])
                        for g in range(CB // 16)]
                for g in range(CB // 16):
                    outt[rb, 0, d, pl.ds(g * 16, 16)] = rows[g]
                return _

            lax.fori_loop(0, D, trans_d, 0, unroll=2)
            pltpu.make_async_copy(
                outt.at[rb], out_hbm.at[pl.ds(tp, 1), j, 0, :,
                                        pl.ds(c0, CB)], sem_ot.at[rb]).start()

    x_copy(0).start()

    def unit0(t, carry):
        phase_a(t)

        @pl.when(t == 1)
        def _():
            phase_b(0, first=True)

        @pl.when(t > 1)
        def _():
            phase_b(t - 1, first=False)
        return carry

    lax.fori_loop(0, T, unit0, 0)
    phase_b(T - 1, first=False)
    for rb in range(NRING):
        out_wait(rb)


def _tc_body(x_ref, w_ref, b_ref, donor_ref, out_ref):
    for j in range(NF):
        xv = x_ref[0, 2 * j + 1, :]
        wv = w_ref[:, j:j + 1]
        bv = b_ref[:, j:j + 1]
        v = xv[None, :] * wv + bv
        out_ref[0, j, 0, :, :] = v * jax.nn.sigmoid(v)


@jax.jit
def kernel(x, tables, W, b):
    xr = jnp.transpose(x, (2, 1, 0))

    sc_run = pl.kernel(
        _sc_body,
        out_type=jax.ShapeDtypeStruct((T, NF, 2, D, B), jnp.float32),
        mesh=plsc.VectorSubcoreMesh(core_axis_name="c", subcore_axis_name="s"),
        compiler_params=pltpu.CompilerParams(
            needs_layout_passes=False, use_tc_tiling_on_sc=False),
        scratch_types=[
            pltpu.VMEM((F, 1, CB), jnp.float32),
            pltpu.VMEM((2, NF, CB), jnp.int32),
            pltpu.VMEM((2, NF, CB, D), jnp.float32),
            pltpu.VMEM((NRING, 1, D, CB), jnp.float32),
            pltpu.SemaphoreType.DMA,
            pltpu.SemaphoreType.DMA((2,)),
            pltpu.SemaphoreType.DMA((NRING,)),
        ],
    )
    outp = sc_run(xr, tables)

    xt = jnp.transpose(x, (1, 2, 0))
    outp = pl.pallas_call(
        _tc_body,
        grid=(T,),
        in_specs=[
            pl.BlockSpec((1, F, B), lambda t: (t, 0, 0)),
            pl.BlockSpec((D, NF), lambda t: (0, 0)),
            pl.BlockSpec((D, NF), lambda t: (0, 0)),
            pl.BlockSpec(memory_space=pl.ANY),
        ],
        out_specs=pl.BlockSpec((1, NF, 1, D, B), lambda t: (t, 0, 1, 0, 0)),
        out_shape=jax.ShapeDtypeStruct((T, NF, 2, D, B), jnp.float32),
        input_output_aliases={3: 0},
    )(xt, W.T, b.T, outp)
    outp = outp.reshape(T, F, D, B)

    return jnp.transpose(outp, (3, 0, 1, 2))
